# baseline (speedup 1.0000x reference)
import jax, jax.numpy as jnp
from jax import lax
from jax.experimental import pallas as pl
from jax.experimental.pallas import tpu as pltpu
from jax.experimental.pallas import tpu_sc as plsc

_C = 100000
_CZ = 2048
_NMAIN = 48
_TAIL = _C - _NMAIN * _CZ
_NG = 2
_NBUF = 4
_LAG = 2


def _tc_copy_block(x_ref, out_ref):
    out_ref[...] = x_ref[...]


def _sc_copy(x_hbm, out_hbm, bufs, tbuf, in_sems, out_sems, tsem):
    nc = 2
    wid = lax.axis_index("s") * nc + lax.axis_index("c")
    base = 512 + wid * (_NG * 8)

    def in_desc(i, slot):
        g = i // _NMAIN
        c = lax.rem(i, _NMAIN)
        return pltpu.make_async_copy(
            x_hbm.at[pl.ds(base + g * 8, 8), pl.ds(c * _CZ, _CZ)],
            bufs.at[slot], in_sems.at[slot])

    def out_desc(i, slot):
        g = i // _NMAIN
        c = lax.rem(i, _NMAIN)
        return pltpu.make_async_copy(
            bufs.at[slot], out_hbm.at[pl.ds(g * 8 + (base - 512), 8), pl.ds(c * _CZ, _CZ)],
            out_sems.at[slot])

    n = _NG * _NMAIN

    def step(it, carry):
        @pl.when(it < n)
        def _():
            slot = lax.rem(it, _NBUF)
            @pl.when(it >= _NBUF)
            def _():
                out_desc(it - _NBUF, slot).wait()
            in_desc(it, slot).start()

        j = it - _LAG

        @pl.when(jnp.logical_and(j >= 0, j < n))
        def _():
            jslot = lax.rem(j, _NBUF)
            in_desc(j, jslot).wait()
            out_desc(j, jslot).start()
        return carry

    lax.fori_loop(0, n + _LAG, step, 0)

    def drain(k, carry):
        i = n - _NBUF + k
        out_desc(i, lax.rem(i, _NBUF)).wait()
        return carry

    lax.fori_loop(0, _NBUF, drain, 0)

    for g in range(_NG):
        src = x_hbm.at[pl.ds(base + g * 8, 8), pl.ds(_NMAIN * _CZ, _TAIL)]
        dst = out_hbm.at[pl.ds(g * 8 + (base - 512), 8), pl.ds(_NMAIN * _CZ, _TAIL)]
        pltpu.make_async_copy(src, tbuf, tsem).start()
        pltpu.make_async_copy(src, tbuf, tsem).wait()
        pltpu.make_async_copy(tbuf, dst, tsem).start()
        pltpu.make_async_copy(tbuf, dst, tsem).wait()


@jax.jit
def _run(teacher_logits, true_labels):
    b, c = teacher_logits.shape
    mesh = plsc.VectorSubcoreMesh(core_axis_name="c", subcore_axis_name="s")
    sc_out = pl.kernel(
        _sc_copy,
        out_type=jax.ShapeDtypeStruct((512, c), jnp.float32),
        mesh=mesh,
        scratch_types=[
            pltpu.VMEM((_NBUF, 8, _CZ), jnp.float32),
            pltpu.VMEM((8, _TAIL), jnp.float32),
            pltpu.SemaphoreType.DMA((_NBUF,)),
            pltpu.SemaphoreType.DMA((_NBUF,)),
            pltpu.SemaphoreType.DMA,
        ],
    )(teacher_logits)
    tc_out = pl.pallas_call(
        _tc_copy_block,
        grid=(512 // 8,),
        in_specs=[pl.BlockSpec((8, c), lambda i: (i, 0))],
        out_specs=pl.BlockSpec((8, c), lambda i: (i, 0)),
        out_shape=jax.ShapeDtypeStruct((512, c), jnp.float32),
    )(teacher_logits[:512])
    return tc_out, sc_out, jnp.ones((b,), jnp.float32)


def kernel(teacher_logits, true_labels):
    return _run(teacher_logits, true_labels)


# speedup vs baseline: 1.1488x; 1.1488x over previous
import functools

import jax
import jax.numpy as jnp
from jax.experimental import pallas as pl

_TEMP = 4.0


def _calibrate_block(labels_ref, x_ref, out_ref, temp_ref):
    x = x_ref[...]
    labels = labels_ref[...]
    pred = jnp.argmax(x, axis=1).astype(jnp.int32)[:, None]
    match = pred == labels
    iota = jax.lax.broadcasted_iota(jnp.int32, x.shape, 1)
    onehot = (iota == labels).astype(x.dtype)
    out_ref[...] = jnp.where(match, x, onehot)
    temp_ref[...] = jnp.where(match, jnp.float32(_TEMP), jnp.float32(1.0))


@functools.partial(jax.jit, static_argnames=("row_block",))
def _calibrate(teacher_logits, true_labels, row_block=16):
    b, c = teacher_logits.shape
    labels2d = true_labels.reshape(b, 1)
    grid = (b // row_block,)
    out, temp = pl.pallas_call(
        _calibrate_block,
        grid=grid,
        in_specs=[
            pl.BlockSpec((row_block, 1), lambda i: (i, 0)),
            pl.BlockSpec((row_block, c), lambda i: (i, 0)),
        ],
        out_specs=[
            pl.BlockSpec((row_block, c), lambda i: (i, 0)),
            pl.BlockSpec((row_block, 1), lambda i: (i, 0)),
        ],
        out_shape=[
            jax.ShapeDtypeStruct((b, c), teacher_logits.dtype),
            jax.ShapeDtypeStruct((b, 1), jnp.float32),
        ],
    )(labels2d, teacher_logits)
    return out, temp.reshape(b)


def kernel(teacher_logits, true_labels):
    return _calibrate(teacher_logits, true_labels)
